# Initial kernel scaffold; baseline (speedup 1.0000x reference)
import functools

import jax
import jax.numpy as jnp
from jax import lax
from jax.experimental import pallas as pl
from jax.experimental.pallas import tpu as pltpu
from jax.experimental.pallas import tpu_sc as plsc

N = 1024
LANES = 16
ROW_W = 128
CHUNK_ROWS = 20
NUM_SUBCORES = 16
NUM_CORES = 2
CHUNKS_PER_SUBCORE = 50
SAMPLES_PER_SUBCORE = CHUNK_ROWS * ROW_W * CHUNKS_PER_SUBCORE
PADDED = SAMPLES_PER_SUBCORE * NUM_SUBCORES
TOTAL_ROWS = PADDED // ROW_W
ROWS_PER_SUBCORE = TOTAL_ROWS // NUM_SUBCORES
REAL_BINS = N * N
HIST_BINS = 1064960
ZERO_PER_SUBCORE = HIST_BINS // NUM_SUBCORES
OUT_PER_SUBCORE = REAL_BINS // NUM_SUBCORES


def _sc_body(ac_hbm, b_hbm, ones_hbm, zeros_hbm, out_hbm,
             idx_v, b_v, ones_v, hist_sh):
    core = lax.axis_index("c")
    sub = lax.axis_index("s")

    pltpu.sync_copy(ones_hbm, ones_v)

    zbase = sub * ZERO_PER_SUBCORE
    pltpu.sync_copy(zeros_hbm.at[pl.ds(zbase, ZERO_PER_SUBCORE)],
                    hist_sh.at[pl.ds(zbase, ZERO_PER_SUBCORE)])
    plsc.subcore_barrier()

    def chunk_body(chunk, carry):
        row_base = sub * ROWS_PER_SUBCORE + chunk * CHUNK_ROWS
        pltpu.sync_copy(ac_hbm.at[core, pl.ds(row_base, CHUNK_ROWS)], idx_v)
        pltpu.sync_copy(b_hbm.at[pl.ds(row_base, CHUNK_ROWS)], b_v)
        for r in range(CHUNK_ROWS):
            for j in range(ROW_W // LANES):
                sl = pl.ds(j * LANES, LANES)
                idx_v[r, sl] = idx_v[r, sl] * N + b_v[r, sl]
        for r in range(CHUNK_ROWS):
            pltpu.sync_copy(ones_v, hist_sh.at[idx_v.at[r]], add=True)
        return carry

    lax.fori_loop(0, CHUNKS_PER_SUBCORE, chunk_body, 0)
    plsc.subcore_barrier()

    obase = sub * OUT_PER_SUBCORE
    pltpu.sync_copy(hist_sh.at[pl.ds(obase, OUT_PER_SUBCORE)],
                    out_hbm.at[core, pl.ds(obase, OUT_PER_SUBCORE)])


_sc_hist = functools.partial(
    pl.kernel,
    out_type=jax.ShapeDtypeStruct((NUM_CORES, REAL_BINS, 1), jnp.float32),
    mesh=plsc.VectorSubcoreMesh(core_axis_name="c", subcore_axis_name="s"),
    scratch_types=[
        pltpu.VMEM((CHUNK_ROWS, ROW_W), jnp.int32),
        pltpu.VMEM((CHUNK_ROWS, ROW_W), jnp.int32),
        pltpu.VMEM((ROW_W, 1), jnp.float32),
        pltpu.VMEM_SHARED((HIST_BINS, 1), jnp.float32),
    ],
)(_sc_body)


def _tc_norm_body(ab_ref, cb_ref, o_ref, *, num_samples):
    ab = ab_ref[...]
    cb = cb_ref[...]
    pib = jnp.sum(ab, axis=0, keepdims=True) * (1.0 / num_samples)
    abn = ab / jnp.maximum(jnp.sum(ab, axis=1, keepdims=True), 1.0)
    cbn = cb / jnp.maximum(jnp.sum(cb, axis=1, keepdims=True), 1.0)
    o_ref[...] = jnp.concatenate([pib, abn, cbn], axis=0)


def kernel(inputs):
    num_samples = inputs.shape[0]
    pad = PADDED - num_samples
    a = jnp.concatenate([inputs[:, 0], jnp.full((pad,), N, jnp.int32)])
    b = jnp.concatenate([inputs[:, 1], jnp.zeros((pad,), jnp.int32)])
    c = jnp.concatenate([inputs[:, 2], jnp.full((pad,), N, jnp.int32)])
    ac = jnp.stack([a, c]).reshape(NUM_CORES, TOTAL_ROWS, ROW_W)
    b2 = b.reshape(TOTAL_ROWS, ROW_W)
    ones_in = jnp.ones((ROW_W, 1), jnp.float32)
    zeros_in = jnp.zeros((HIST_BINS, 1), jnp.float32)

    counts = _sc_hist(ac, b2, ones_in, zeros_in)
    ab = counts[0].reshape(N, N)
    cb = counts[1].reshape(N, N)

    out = pl.pallas_call(
        functools.partial(_tc_norm_body, num_samples=float(num_samples)),
        out_shape=jax.ShapeDtypeStruct((2 * N + 1, N), jnp.float32),
    )(ab, cb)
    return out


# speedup vs baseline: 25.7111x; 25.7111x over previous
import functools

import jax
import jax.numpy as jnp
from jax import lax
from jax.experimental import pallas as pl
from jax.experimental.pallas import tpu as pltpu
from jax.experimental.pallas import tpu_sc as plsc

N = 1024
LANES = 16
ROW_W = 128
CHUNK_ROWS = 16
NUM_SUBCORES = 16
NUM_CORES = 2
CHUNKS_PER_SUBCORE = 63
SAMPLES_PER_SUBCORE = CHUNK_ROWS * ROW_W * CHUNKS_PER_SUBCORE
PADDED = SAMPLES_PER_SUBCORE * NUM_SUBCORES
TOTAL_ROWS = PADDED // ROW_W
ROWS_PER_SUBCORE = TOTAL_ROWS // NUM_SUBCORES
REAL_BINS = N * N
HIST_BINS = 1064960
ZERO_PER_SUBCORE = HIST_BINS // NUM_SUBCORES
OUT_PER_SUBCORE = REAL_BINS // NUM_SUBCORES


def _sc_body(ac_hbm, b_hbm, zeros_hbm, out_hbm, idx_v, b_v, ones_v, hist_sh):
    core = lax.axis_index("c")
    sub = lax.axis_index("s")

    for i in range(ROW_W // LANES):
        ones_v[pl.ds(i * LANES, LANES)] = jnp.ones((LANES,), jnp.float32)

    zbase = sub * ZERO_PER_SUBCORE
    pltpu.sync_copy(zeros_hbm.at[pl.ds(zbase, ZERO_PER_SUBCORE)],
                    hist_sh.at[pl.ds(zbase, ZERO_PER_SUBCORE)])
    plsc.subcore_barrier()

    def chunk_body(chunk, carry):
        row_base = sub * ROWS_PER_SUBCORE + chunk * CHUNK_ROWS
        pltpu.sync_copy(ac_hbm.at[core, pl.ds(row_base, CHUNK_ROWS)], idx_v)
        pltpu.sync_copy(b_hbm.at[pl.ds(row_base, CHUNK_ROWS)], b_v)
        for r in range(CHUNK_ROWS):
            for j in range(ROW_W // LANES):
                sl = pl.ds(j * LANES, LANES)
                idx_v[r, sl] = idx_v[r, sl] * N + b_v[r, sl]
        for r in range(CHUNK_ROWS):
            pltpu.sync_copy(ones_v, hist_sh.at[idx_v.at[r]], add=True)
        return carry

    lax.fori_loop(0, CHUNKS_PER_SUBCORE, chunk_body, 0)
    plsc.subcore_barrier()

    obase = sub * OUT_PER_SUBCORE
    pltpu.sync_copy(hist_sh.at[pl.ds(obase, OUT_PER_SUBCORE)],
                    out_hbm.at[core, pl.ds(obase, OUT_PER_SUBCORE)])


_sc_hist = functools.partial(
    pl.kernel,
    out_type=jax.ShapeDtypeStruct((NUM_CORES, REAL_BINS), jnp.float32),
    mesh=plsc.VectorSubcoreMesh(core_axis_name="c", subcore_axis_name="s"),
    scratch_types=[
        pltpu.VMEM((CHUNK_ROWS, ROW_W), jnp.int32),
        pltpu.VMEM((CHUNK_ROWS, ROW_W), jnp.int32),
        pltpu.VMEM((ROW_W,), jnp.float32),
        pltpu.VMEM_SHARED((HIST_BINS,), jnp.float32),
    ],
)(_sc_body)


def _tc_norm_body(ab_ref, cb_ref, o_ref, *, num_samples):
    ab = ab_ref[...]
    cb = cb_ref[...]
    pib = jnp.sum(ab, axis=0, keepdims=True) * (1.0 / num_samples)
    abn = ab / jnp.maximum(jnp.sum(ab, axis=1, keepdims=True), 1.0)
    cbn = cb / jnp.maximum(jnp.sum(cb, axis=1, keepdims=True), 1.0)
    o_ref[...] = jnp.concatenate([pib, abn, cbn], axis=0)


def kernel(inputs):
    num_samples = inputs.shape[0]
    pad = PADDED - num_samples
    pad_b = (jnp.arange(pad, dtype=jnp.int32)) % N
    a = jnp.concatenate([inputs[:, 0], jnp.full((pad,), N, jnp.int32)])
    b = jnp.concatenate([inputs[:, 1], pad_b])
    c = jnp.concatenate([inputs[:, 2], jnp.full((pad,), N, jnp.int32)])
    ac = jnp.stack([a, c]).reshape(NUM_CORES, TOTAL_ROWS, ROW_W)
    b2 = b.reshape(TOTAL_ROWS, ROW_W)
    zeros_in = jnp.zeros((HIST_BINS,), jnp.float32)

    counts = _sc_hist(ac, b2, zeros_in)
    ab = counts[0].reshape(N, N)
    cb = counts[1].reshape(N, N)

    out = pl.pallas_call(
        functools.partial(_tc_norm_body, num_samples=float(num_samples)),
        out_shape=jax.ShapeDtypeStruct((2 * N + 1, N), jnp.float32),
    )(ab, cb)
    return out


# speedup vs baseline: 41.4180x; 1.6109x over previous
import functools

import jax
import jax.numpy as jnp
from jax import lax
from jax.experimental import pallas as pl
from jax.experimental.pallas import tpu as pltpu
from jax.experimental.pallas import tpu_sc as plsc

N = 1024
LANES = 16
ROW_W = 128
CHUNK_ROWS = 16
NUM_SUBCORES = 16
NUM_CORES = 2
CHUNKS_PER_SUBCORE = 63
SAMPLES_PER_SUBCORE = CHUNK_ROWS * ROW_W * CHUNKS_PER_SUBCORE
PADDED = SAMPLES_PER_SUBCORE * NUM_SUBCORES
TOTAL_ROWS = PADDED // ROW_W
ROWS_PER_SUBCORE = TOTAL_ROWS // NUM_SUBCORES
REAL_BINS = N * N
HIST_BINS = 1064960
ZERO_PER_SUBCORE = HIST_BINS // NUM_SUBCORES
OUT_PER_SUBCORE = REAL_BINS // NUM_SUBCORES


NBUF = 3


def _sc_body(ac_hbm, b_hbm, zeros_hbm, out_hbm,
             idx_v, b_v, ones_v, hist_sh, sem_in, sem_sc):
    core = lax.axis_index("c")
    sub = lax.axis_index("s")

    for i in range(ROW_W // LANES):
        ones_v[pl.ds(i * LANES, LANES)] = jnp.ones((LANES,), jnp.float32)

    zbase = sub * ZERO_PER_SUBCORE
    pltpu.sync_copy(zeros_hbm.at[pl.ds(zbase, ZERO_PER_SUBCORE)],
                    hist_sh.at[pl.ds(zbase, ZERO_PER_SUBCORE)])
    plsc.subcore_barrier()

    base_row = sub * ROWS_PER_SUBCORE

    def start_in(g, p):
        rb = base_row + g * CHUNK_ROWS
        pltpu.async_copy(ac_hbm.at[core, pl.ds(rb, CHUNK_ROWS)],
                         idx_v.at[p], sem_in.at[p])
        pltpu.async_copy(b_hbm.at[pl.ds(rb, CHUNK_ROWS)],
                         b_v.at[p], sem_in.at[p])

    def wait_in(p):
        pltpu.make_async_copy(ac_hbm.at[core, pl.ds(0, CHUNK_ROWS)],
                              idx_v.at[p], sem_in.at[p]).wait()
        pltpu.make_async_copy(b_hbm.at[pl.ds(0, CHUNK_ROWS)],
                              b_v.at[p], sem_in.at[p]).wait()

    def fire_sc(p):
        for r in range(CHUNK_ROWS):
            pltpu.async_copy(ones_v, hist_sh.at[idx_v.at[p].at[r]],
                             sem_sc.at[p], add=True)

    def drain_sc(p):
        for r in range(CHUNK_ROWS):
            pltpu.make_async_copy(ones_v, hist_sh.at[idx_v.at[p].at[r]],
                                  sem_sc.at[p]).wait()

    start_in(0, 0)

    def iter_body(it, carry):
        g0 = it * NBUF
        for p in range(NBUF):
            g = g0 + p
            @pl.when(g >= 2)
            def _():
                drain_sc((p + 1) % NBUF)

            @pl.when(g + 1 < CHUNKS_PER_SUBCORE)
            def _():
                start_in(g + 1, (p + 1) % NBUF)

            wait_in(p)
            for r in range(CHUNK_ROWS):
                for j in range(ROW_W // LANES):
                    sl = pl.ds(j * LANES, LANES)
                    idx_v[p, r, sl] = idx_v[p, r, sl] * N + b_v[p, r, sl]
            fire_sc(p)
        return carry

    lax.fori_loop(0, CHUNKS_PER_SUBCORE // NBUF, iter_body, 0)
    drain_sc((CHUNKS_PER_SUBCORE - 2) % NBUF)
    drain_sc((CHUNKS_PER_SUBCORE - 1) % NBUF)
    plsc.subcore_barrier()

    obase = sub * OUT_PER_SUBCORE
    pltpu.sync_copy(hist_sh.at[pl.ds(obase, OUT_PER_SUBCORE)],
                    out_hbm.at[core, pl.ds(obase, OUT_PER_SUBCORE)])


_sc_hist = functools.partial(
    pl.kernel,
    out_type=jax.ShapeDtypeStruct((NUM_CORES, REAL_BINS), jnp.float32),
    mesh=plsc.VectorSubcoreMesh(core_axis_name="c", subcore_axis_name="s"),
    scratch_types=[
        pltpu.VMEM((NBUF, CHUNK_ROWS, ROW_W), jnp.int32),
        pltpu.VMEM((NBUF, CHUNK_ROWS, ROW_W), jnp.int32),
        pltpu.VMEM((ROW_W,), jnp.float32),
        pltpu.VMEM_SHARED((HIST_BINS,), jnp.float32),
        pltpu.SemaphoreType.DMA((NBUF,)),
        pltpu.SemaphoreType.DMA((NBUF,)),
    ],
)(_sc_body)


def _tc_norm_body(ab_ref, cb_ref, o_ref, *, num_samples):
    ab = ab_ref[...]
    cb = cb_ref[...]
    pib = jnp.sum(ab, axis=0, keepdims=True) * (1.0 / num_samples)
    abn = ab / jnp.maximum(jnp.sum(ab, axis=1, keepdims=True), 1.0)
    cbn = cb / jnp.maximum(jnp.sum(cb, axis=1, keepdims=True), 1.0)
    o_ref[...] = jnp.concatenate([pib, abn, cbn], axis=0)


def kernel(inputs):
    num_samples = inputs.shape[0]
    pad = PADDED - num_samples
    pad_b = (jnp.arange(pad, dtype=jnp.int32)) % N
    a = jnp.concatenate([inputs[:, 0], jnp.full((pad,), N, jnp.int32)])
    b = jnp.concatenate([inputs[:, 1], pad_b])
    c = jnp.concatenate([inputs[:, 2], jnp.full((pad,), N, jnp.int32)])
    ac = jnp.stack([a, c]).reshape(NUM_CORES, TOTAL_ROWS, ROW_W)
    b2 = b.reshape(TOTAL_ROWS, ROW_W)
    zeros_in = jnp.zeros((HIST_BINS,), jnp.float32)

    counts = _sc_hist(ac, b2, zeros_in)
    ab = counts[0].reshape(N, N)
    cb = counts[1].reshape(N, N)

    out = pl.pallas_call(
        functools.partial(_tc_norm_body, num_samples=float(num_samples)),
        out_shape=jax.ShapeDtypeStruct((2 * N + 1, N), jnp.float32),
    )(ab, cb)
    return out
